# Initial kernel scaffold; baseline (speedup 1.0000x reference)
import functools

import jax
import jax.numpy as jnp
from jax import lax
from jax.experimental import pallas as pl
from jax.experimental.pallas import tpu as pltpu
from jax.experimental.pallas import tpu_sc as plsc

N = 10000
E = 320000
D = 128
UNIT = 128
R = E // UNIT
NSC = 2
NSUB = 16
NW = NSC * NSUB
NPB = 400
EPB = 1600

_relu = jax.nn.relu


def _prep_body(x_ref, posp_ref, WinT, b_in, WlinT, WsrcT, WdstT, P1pT, pb1,
               P2T, pb2, A1T, ab1, A2T, ab2,
               T1_ref, T2_ref, den0_ref, num0_ref):
    x = x_ref[...]
    h = _relu(jnp.dot(x, WinT[...], preferred_element_type=jnp.float32)
              + b_in[...])
    xl = jnp.dot(h, WlinT[...], preferred_element_type=jnp.float32)
    dd = jnp.dot(jnp.dot(h, WdstT[...], preferred_element_type=jnp.float32),
                 A1T[...], preferred_element_type=jnp.float32)
    ss = jnp.dot(jnp.dot(h, WsrcT[...], preferred_element_type=jnp.float32),
                 A1T[...], preferred_element_type=jnp.float32)
    q = jnp.dot(posp_ref[...], P1pT[...], preferred_element_type=jnp.float32)
    dl64 = _relu(pb1[...])
    dl128 = _relu(jnp.dot(dl64, P2T[...],
                          preferred_element_type=jnp.float32) + pb2[...])
    v0 = _relu(jnp.dot(dl128, A1T[...], preferred_element_type=jnp.float32)
               + dd - ss + ab1[...])
    alpha0 = _relu(jnp.dot(v0, A2T[...], preferred_element_type=jnp.float32)
                   + ab2[...])
    ex0 = jnp.exp(alpha0)
    den0_ref[...] = ex0
    num0_ref[...] = ex0 * (xl + dl128)
    T1_ref[...] = jnp.concatenate([q, dd], axis=1)
    T2_ref[...] = jnp.concatenate([q, ss, xl], axis=1)


def _prep_call(x, posp, WinT, b_in, WlinT, WsrcT, WdstT, P1pT, pb1, P2T, pb2,
               A1T, ab1, A2T, ab2):
    nb = N // NPB
    full = lambda a: pl.BlockSpec(a.shape, lambda i: (0,) * a.ndim)
    row_spec = lambda w: pl.BlockSpec((NPB, w), lambda i: (i, 0))
    return pl.pallas_call(
        _prep_body,
        grid=(nb,),
        in_specs=[row_spec(D), row_spec(8)] + [
            full(a) for a in (WinT, b_in, WlinT, WsrcT, WdstT, P1pT, pb1,
                              P2T, pb2, A1T, ab1, A2T, ab2)],
        out_specs=[row_spec(D), row_spec(2 * D), row_spec(D), row_spec(D)],
        out_shape=[jax.ShapeDtypeStruct((N, D), jnp.float32),
                   jax.ShapeDtypeStruct((N, 2 * D), jnp.float32),
                   jax.ShapeDtypeStruct((N, D), jnp.float32),
                   jax.ShapeDtypeStruct((N, D), jnp.float32)],
    )(x, posp, WinT, b_in, WlinT, WsrcT, WdstT, P1pT, pb1, P2T, pb2,
      A1T, ab1, A2T, ab2)


_sc_mesh = plsc.VectorSubcoreMesh(core_axis_name="c", subcore_axis_name="s")


@functools.partial(
    pl.kernel,
    mesh=_sc_mesh,
    out_type=[jax.ShapeDtypeStruct((E, D), jnp.float32),
              jax.ShapeDtypeStruct((E, 2 * D), jnp.float32)],
    scratch_types=[pltpu.VMEM((UNIT,), jnp.int32),
                   pltpu.VMEM((UNIT,), jnp.int32),
                   pltpu.VMEM((UNIT, D), jnp.float32),
                   pltpu.VMEM((UNIT, 2 * D), jnp.float32),
                   pltpu.SemaphoreType.DMA],
)
def _sc_gather(src2d, dst2d, T1, T2, outD, outS, idx_s, idx_d, bufD, bufS,
               sem):
    c = lax.axis_index("c")
    s = lax.axis_index("s")
    wid = s * NSC + c
    nrows = 78 + jnp.where(wid < R - 78 * NW, 1, 0)

    def body(i, carry):
        row = wid + i * NW
        pltpu.sync_copy(src2d.at[row], idx_s)
        pltpu.sync_copy(dst2d.at[row], idx_d)
        cp1 = pltpu.async_copy(T1.at[idx_d], bufD, sem)
        cp2 = pltpu.async_copy(T2.at[idx_s], bufS, sem)
        cp1.wait()
        cp2.wait()
        pltpu.sync_copy(bufD, outD.at[pl.ds(row * UNIT, UNIT), :])
        pltpu.sync_copy(bufS, outS.at[pl.ds(row * UNIT, UNIT), :])
        return carry

    lax.fori_loop(0, nrows, body, 0)


def _emlp_body(eD_ref, eS_ref, pb1, P2T, pb2, A1T, ab1, A2T, ab2, out_ref):
    eD = eD_ref[...]
    eS = eS_ref[...]
    e1 = _relu(eD[:, :64] - eS[:, :64] + pb1[...])
    delta = _relu(jnp.dot(e1, P2T[...], preferred_element_type=jnp.float32)
                  + pb2[...])
    v = _relu(jnp.dot(delta, A1T[...], preferred_element_type=jnp.float32)
              + (eD[:, 64:] - eS[:, 64:128]) + ab1[...])
    alpha = _relu(jnp.dot(v, A2T[...], preferred_element_type=jnp.float32)
                  + ab2[...])
    ex = jnp.exp(alpha)
    out_ref[:, :D] = ex
    out_ref[:, D:] = ex * (eS[:, 128:] + delta)


def _emlp_call(eD, eS, pb1, P2T, pb2, A1T, ab1, A2T, ab2):
    nb = E // EPB
    full = lambda a: pl.BlockSpec(a.shape, lambda i: (0,) * a.ndim)
    return pl.pallas_call(
        _emlp_body,
        grid=(nb,),
        in_specs=[pl.BlockSpec((EPB, D), lambda i: (i, 0)),
                  pl.BlockSpec((EPB, 2 * D), lambda i: (i, 0))] + [
            full(a) for a in (pb1, P2T, pb2, A1T, ab1, A2T, ab2)],
        out_specs=pl.BlockSpec((EPB, 2 * D), lambda i: (i, 0)),
        out_shape=jax.ShapeDtypeStruct((E, 2 * D), jnp.float32),
    )(eD, eS, pb1, P2T, pb2, A1T, ab1, A2T, ab2)


_ROWS_PER_SUB = N // NSUB


@functools.partial(
    pl.kernel,
    mesh=_sc_mesh,
    out_type=[jax.ShapeDtypeStruct((N, D), jnp.float32),
              jax.ShapeDtypeStruct((N, D), jnp.float32)],
    scratch_types=[pltpu.VMEM((UNIT,), jnp.int32),
                   pltpu.VMEM((UNIT, 64), jnp.float32),
                   pltpu.VMEM((UNIT, 64), jnp.float32),
                   pltpu.VMEM_SHARED((N, 64), jnp.float32),
                   pltpu.VMEM_SHARED((N, 64), jnp.float32)],
)
def _sc_scatter(dst2d, em, zeros, denO, numO, idx_d, exb, mb, den_sh, num_sh):
    c = lax.axis_index("c")
    s = lax.axis_index("s")
    r0 = s * _ROWS_PER_SUB
    pltpu.sync_copy(zeros, den_sh.at[pl.ds(r0, _ROWS_PER_SUB), :])
    pltpu.sync_copy(zeros, num_sh.at[pl.ds(r0, _ROWS_PER_SUB), :])
    plsc.subcore_barrier()
    nrows = 156 + jnp.where(s < R - 156 * NSUB, 1, 0)
    ccol = c * 64

    def body(i, carry):
        row = s + i * NSUB
        pltpu.sync_copy(dst2d.at[row], idx_d)
        pltpu.sync_copy(em.at[pl.ds(row * UNIT, UNIT), pl.ds(ccol, 64)], exb)
        pltpu.sync_copy(em.at[pl.ds(row * UNIT, UNIT), pl.ds(ccol + D, 64)],
                        mb)
        pltpu.sync_copy(exb, den_sh.at[idx_d], add=True)
        pltpu.sync_copy(mb, num_sh.at[idx_d], add=True)
        return carry

    lax.fori_loop(0, nrows, body, 0)
    plsc.subcore_barrier()
    pltpu.sync_copy(den_sh.at[pl.ds(r0, _ROWS_PER_SUB), :],
                    denO.at[pl.ds(r0, _ROWS_PER_SUB), pl.ds(ccol, 64)])
    pltpu.sync_copy(num_sh.at[pl.ds(r0, _ROWS_PER_SUB), :],
                    numO.at[pl.ds(r0, _ROWS_PER_SUB), pl.ds(ccol, 64)])


def _final_body(den_ref, num_ref, den0_ref, num0_ref, WoutT, b_out, o_ref):
    den = den_ref[...] + den0_ref[...] + 1e-16
    num = num_ref[...] + num0_ref[...]
    o_ref[...] = _relu(jnp.dot(num / den, WoutT[...],
                               preferred_element_type=jnp.float32)
                       + b_out[...])


def _final_call(denE, numE, den0, num0, WoutT, b_out):
    nb = N // NPB
    full = lambda a: pl.BlockSpec(a.shape, lambda i: (0,) * a.ndim)
    row_spec = pl.BlockSpec((NPB, D), lambda i: (i, 0))
    return pl.pallas_call(
        _final_body,
        grid=(nb,),
        in_specs=[row_spec, row_spec, row_spec, row_spec, full(WoutT),
                  full(b_out)],
        out_specs=row_spec,
        out_shape=jax.ShapeDtypeStruct((N, D), jnp.float32),
    )(denE, numE, den0, num0, WoutT, b_out)


def kernel(x, pos, edge_index, Win, b_in, Wout, b_out, Wlin, Wsrc, Wdst,
           P1, pb1, P2, pb2, A1, ab1, A2, ab2):
    posp = jnp.pad(pos, ((0, 0), (0, 5)))
    P1pT = jnp.pad(P1, ((0, 0), (0, 5))).T
    row = lambda v: v.reshape(1, -1)
    T1, T2, den0, num0 = _prep_call(
        x, posp, Win.T, row(b_in), Wlin.T, Wsrc.T, Wdst.T, P1pT, row(pb1),
        P2.T, row(pb2), A1.T, row(ab1), A2.T, row(ab2))
    src2d = edge_index[0].reshape(R, UNIT)
    dst2d = edge_index[1].reshape(R, UNIT)
    eD, eS = _sc_gather(src2d, dst2d, T1, T2)
    em = _emlp_call(eD, eS, row(pb1), P2.T, row(pb2), A1.T, row(ab1),
                    A2.T, row(ab2))
    zeros = jnp.zeros((_ROWS_PER_SUB, 64), jnp.float32)
    denE, numE = _sc_scatter(dst2d, em, zeros)
    return _final_call(denE, numE, den0, num0, Wout.T, row(b_out))


# speedup vs baseline: 7.5809x; 7.5809x over previous
import functools

import jax
import jax.numpy as jnp
from jax import lax
from jax.experimental import pallas as pl
from jax.experimental.pallas import tpu as pltpu
from jax.experimental.pallas import tpu_sc as plsc

N = 10000
E = 320000
D = 128
UNIT = 128
R = E // UNIT
NSC = 2
NSUB = 16
NW = NSC * NSUB
NPB = 400
EPB = 1600

_relu = jax.nn.relu


def _prep_body(x_ref, posp_ref, WinT, b_in, WlinT, WsrcT, WdstT, P1pT, pb1,
               P2T, pb2, A1T, ab1, A2T, ab2,
               T1_ref, T2_ref, den0_ref, num0_ref):
    x = x_ref[...]
    h = _relu(jnp.dot(x, WinT[...], preferred_element_type=jnp.float32)
              + b_in[...])
    xl = jnp.dot(h, WlinT[...], preferred_element_type=jnp.float32)
    dd = jnp.dot(jnp.dot(h, WdstT[...], preferred_element_type=jnp.float32),
                 A1T[...], preferred_element_type=jnp.float32)
    ss = jnp.dot(jnp.dot(h, WsrcT[...], preferred_element_type=jnp.float32),
                 A1T[...], preferred_element_type=jnp.float32)
    q = jnp.dot(posp_ref[...], P1pT[...], preferred_element_type=jnp.float32)
    dl64 = _relu(pb1[...])
    dl128 = _relu(jnp.dot(dl64, P2T[...],
                          preferred_element_type=jnp.float32) + pb2[...])
    v0 = _relu(jnp.dot(dl128, A1T[...], preferred_element_type=jnp.float32)
               + dd - ss + ab1[...])
    alpha0 = _relu(jnp.dot(v0, A2T[...], preferred_element_type=jnp.float32)
                   + ab2[...])
    ex0 = jnp.exp(alpha0)
    den0_ref[...] = ex0
    num0_ref[...] = ex0 * (xl + dl128)
    T1_ref[...] = jnp.concatenate([q, dd], axis=1)
    T2_ref[...] = jnp.concatenate([q, ss, xl], axis=1)


def _prep_call(x, posp, WinT, b_in, WlinT, WsrcT, WdstT, P1pT, pb1, P2T, pb2,
               A1T, ab1, A2T, ab2):
    nb = N // NPB
    full = lambda a: pl.BlockSpec(a.shape, lambda i: (0,) * a.ndim)
    row_spec = lambda w: pl.BlockSpec((NPB, w), lambda i: (i, 0))
    return pl.pallas_call(
        _prep_body,
        grid=(nb,),
        in_specs=[row_spec(D), row_spec(8)] + [
            full(a) for a in (WinT, b_in, WlinT, WsrcT, WdstT, P1pT, pb1,
                              P2T, pb2, A1T, ab1, A2T, ab2)],
        out_specs=[row_spec(D), row_spec(2 * D), row_spec(D), row_spec(D)],
        out_shape=[jax.ShapeDtypeStruct((N, D), jnp.float32),
                   jax.ShapeDtypeStruct((N, 2 * D), jnp.float32),
                   jax.ShapeDtypeStruct((N, D), jnp.float32),
                   jax.ShapeDtypeStruct((N, D), jnp.float32)],
    )(x, posp, WinT, b_in, WlinT, WsrcT, WdstT, P1pT, pb1, P2T, pb2,
      A1T, ab1, A2T, ab2)


_sc_mesh = plsc.VectorSubcoreMesh(core_axis_name="c", subcore_axis_name="s")


@functools.partial(
    pl.kernel,
    mesh=_sc_mesh,
    out_type=[jax.ShapeDtypeStruct((E, D), jnp.float32),
              jax.ShapeDtypeStruct((E, 2 * D), jnp.float32)],
    scratch_types=[pltpu.VMEM((UNIT,), jnp.int32),
                   pltpu.VMEM((UNIT,), jnp.int32),
                   pltpu.VMEM((UNIT, D), jnp.float32),
                   pltpu.VMEM((UNIT, 2 * D), jnp.float32),
                   pltpu.SemaphoreType.DMA],
)
def _sc_gather(src1, dst1, T1, T2, outD, outS, idx_s, idx_d, bufD, bufS,
               sem):
    c = lax.axis_index("c")
    s = lax.axis_index("s")
    wid = s * NSC + c
    nrows = 78 + jnp.where(wid < R - 78 * NW, 1, 0)

    def body(i, carry):
        row = wid + i * NW
        off = pl.multiple_of(row * UNIT, UNIT)
        pltpu.sync_copy(src1.at[pl.ds(off, UNIT)], idx_s)
        pltpu.sync_copy(dst1.at[pl.ds(off, UNIT)], idx_d)
        cp1 = pltpu.async_copy(T1.at[idx_d], bufD, sem)
        cp2 = pltpu.async_copy(T2.at[idx_s], bufS, sem)
        cp1.wait()
        cp2.wait()
        pltpu.sync_copy(bufD, outD.at[pl.ds(off, UNIT), :])
        pltpu.sync_copy(bufS, outS.at[pl.ds(off, UNIT), :])
        return carry

    lax.fori_loop(0, nrows, body, 0)


def _emlp_body(eD_ref, eS_ref, pb1, P2T, pb2, A1T, ab1, A2T, ab2, out_ref):
    eD = eD_ref[...]
    eS = eS_ref[...]
    e1 = _relu(eD[:, :64] - eS[:, :64] + pb1[...])
    delta = _relu(jnp.dot(e1, P2T[...], preferred_element_type=jnp.float32)
                  + pb2[...])
    v = _relu(jnp.dot(delta, A1T[...], preferred_element_type=jnp.float32)
              + (eD[:, 64:] - eS[:, 64:128]) + ab1[...])
    alpha = _relu(jnp.dot(v, A2T[...], preferred_element_type=jnp.float32)
                  + ab2[...])
    ex = jnp.exp(alpha)
    msg = ex * (eS[:, 128:] + delta)
    out_ref[0] = jnp.concatenate([ex[:, :64], msg[:, :64]], axis=1)
    out_ref[1] = jnp.concatenate([ex[:, 64:], msg[:, 64:]], axis=1)


def _emlp_call(eD, eS, pb1, P2T, pb2, A1T, ab1, A2T, ab2):
    nb = E // EPB
    full = lambda a: pl.BlockSpec(a.shape, lambda i: (0,) * a.ndim)
    return pl.pallas_call(
        _emlp_body,
        grid=(nb,),
        in_specs=[pl.BlockSpec((EPB, D), lambda i: (i, 0)),
                  pl.BlockSpec((EPB, 2 * D), lambda i: (i, 0))] + [
            full(a) for a in (pb1, P2T, pb2, A1T, ab1, A2T, ab2)],
        out_specs=pl.BlockSpec((2, EPB, D), lambda i: (0, i, 0)),
        out_shape=jax.ShapeDtypeStruct((2, E, D), jnp.float32),
    )(eD, eS, pb1, P2T, pb2, A1T, ab1, A2T, ab2)


_RSLICE = 624


@functools.partial(
    pl.kernel,
    mesh=_sc_mesh,
    out_type=jax.ShapeDtypeStruct((2, N, D), jnp.float32),
    scratch_types=[pltpu.VMEM((UNIT,), jnp.int32),
                   pltpu.VMEM((UNIT, D), jnp.float32),
                   pltpu.VMEM_SHARED((N, D), jnp.float32)],
)
def _sc_scatter(dst1, em3, zeros, accO, idx_d, bufE, acc_sh):
    c = lax.axis_index("c")
    s = lax.axis_index("s")
    r0 = s * _RSLICE
    pltpu.sync_copy(zeros, acc_sh.at[pl.ds(r0, _RSLICE), :])

    @pl.when(s == NSUB - 1)
    def _():
        pltpu.sync_copy(zeros.at[pl.ds(0, 16), :],
                        acc_sh.at[pl.ds(NSUB * _RSLICE, 16), :])

    plsc.subcore_barrier()
    nrows = 156 + jnp.where(s < R - 156 * NSUB, 1, 0)

    def body(i, carry):
        row = s + i * NSUB
        off = pl.multiple_of(row * UNIT, UNIT)
        pltpu.sync_copy(dst1.at[pl.ds(off, UNIT)], idx_d)
        pltpu.sync_copy(em3.at[c, pl.ds(off, UNIT), :], bufE)
        pltpu.sync_copy(bufE, acc_sh.at[idx_d], add=True)
        return carry

    lax.fori_loop(0, nrows, body, 0)
    plsc.subcore_barrier()
    pltpu.sync_copy(acc_sh.at[pl.ds(r0, _RSLICE), :],
                    accO.at[c, pl.ds(r0, _RSLICE), :])

    @pl.when(s == NSUB - 1)
    def _():
        pltpu.sync_copy(acc_sh.at[pl.ds(NSUB * _RSLICE, 16), :],
                        accO.at[c, pl.ds(NSUB * _RSLICE, 16), :])


def _final_body(acc_ref, den0_ref, num0_ref, WoutT, b_out, o_ref):
    acc = acc_ref[...]
    den = jnp.concatenate([acc[0, :, :64], acc[1, :, :64]], axis=1)
    num = jnp.concatenate([acc[0, :, 64:], acc[1, :, 64:]], axis=1)
    den = den + den0_ref[...] + 1e-16
    num = num + num0_ref[...]
    o_ref[...] = _relu(jnp.dot(num / den, WoutT[...],
                               preferred_element_type=jnp.float32)
                       + b_out[...])


def _final_call(accE, den0, num0, WoutT, b_out):
    nb = N // NPB
    full = lambda a: pl.BlockSpec(a.shape, lambda i: (0,) * a.ndim)
    row_spec = pl.BlockSpec((NPB, D), lambda i: (i, 0))
    return pl.pallas_call(
        _final_body,
        grid=(nb,),
        in_specs=[pl.BlockSpec((2, NPB, D), lambda i: (0, i, 0)),
                  row_spec, row_spec, full(WoutT), full(b_out)],
        out_specs=row_spec,
        out_shape=jax.ShapeDtypeStruct((N, D), jnp.float32),
    )(accE, den0, num0, WoutT, b_out)


def kernel(x, pos, edge_index, Win, b_in, Wout, b_out, Wlin, Wsrc, Wdst,
           P1, pb1, P2, pb2, A1, ab1, A2, ab2):
    posp = jnp.pad(pos, ((0, 0), (0, 5)))
    P1pT = jnp.pad(P1, ((0, 0), (0, 5))).T
    row = lambda v: v.reshape(1, -1)
    T1, T2, den0, num0 = _prep_call(
        x, posp, Win.T, row(b_in), Wlin.T, Wsrc.T, Wdst.T, P1pT, row(pb1),
        P2.T, row(pb2), A1.T, row(ab1), A2.T, row(ab2))
    src1 = edge_index[0]
    dst1 = edge_index[1]
    eD, eS = _sc_gather(src1, dst1, T1, T2)
    em3 = _emlp_call(eD, eS, row(pb1), P2.T, row(pb2), A1.T, row(ab1),
                     A2.T, row(ab2))
    zeros = jnp.zeros((_RSLICE, D), jnp.float32)
    accE = _sc_scatter(dst1, em3, zeros)
    return _final_call(accE, den0, num0, Wout.T, row(b_out))


# speedup vs baseline: 8.1536x; 1.0755x over previous
import functools

import jax
import jax.numpy as jnp
from jax import lax
from jax.experimental import pallas as pl
from jax.experimental.pallas import tpu as pltpu
from jax.experimental.pallas import tpu_sc as plsc

N = 10000
E = 320000
D = 128
UNIT = 128
R = E // UNIT
NSC = 2
NSUB = 16
NW = NSC * NSUB
NPB = 400
EPB = 1600
K = 4
EC = E // K
RC = EC // UNIT

_relu = jax.nn.relu


def _prep_body(x_ref, posp_ref, WinT, b_in, WlinT, WsrcT, WdstT, P1pT, pb1,
               P2T, pb2, A1T, ab1, A2T, ab2,
               T1_ref, T2_ref, den0_ref, num0_ref):
    x = x_ref[...]
    h = _relu(jnp.dot(x, WinT[...], preferred_element_type=jnp.float32)
              + b_in[...])
    xl = jnp.dot(h, WlinT[...], preferred_element_type=jnp.float32)
    dd = jnp.dot(jnp.dot(h, WdstT[...], preferred_element_type=jnp.float32),
                 A1T[...], preferred_element_type=jnp.float32)
    ss = jnp.dot(jnp.dot(h, WsrcT[...], preferred_element_type=jnp.float32),
                 A1T[...], preferred_element_type=jnp.float32)
    q = jnp.dot(posp_ref[...], P1pT[...], preferred_element_type=jnp.float32)
    dl64 = _relu(pb1[...])
    dl128 = _relu(jnp.dot(dl64, P2T[...],
                          preferred_element_type=jnp.float32) + pb2[...])
    v0 = _relu(jnp.dot(dl128, A1T[...], preferred_element_type=jnp.float32)
               + dd - ss + ab1[...])
    alpha0 = _relu(jnp.dot(v0, A2T[...], preferred_element_type=jnp.float32)
                   + ab2[...])
    ex0 = jnp.exp(alpha0)
    den0_ref[...] = ex0
    num0_ref[...] = ex0 * (xl + dl128)
    T1_ref[...] = jnp.concatenate([q, dd], axis=1)
    T2_ref[...] = jnp.concatenate([q, ss, xl], axis=1)


def _prep_call(x, posp, WinT, b_in, WlinT, WsrcT, WdstT, P1pT, pb1, P2T, pb2,
               A1T, ab1, A2T, ab2):
    nb = N // NPB
    full = lambda a: pl.BlockSpec(a.shape, lambda i: (0,) * a.ndim)
    row_spec = lambda w: pl.BlockSpec((NPB, w), lambda i: (i, 0))
    return pl.pallas_call(
        _prep_body,
        grid=(nb,),
        in_specs=[row_spec(D), row_spec(8)] + [
            full(a) for a in (WinT, b_in, WlinT, WsrcT, WdstT, P1pT, pb1,
                              P2T, pb2, A1T, ab1, A2T, ab2)],
        out_specs=[row_spec(D), row_spec(2 * D), row_spec(D), row_spec(D)],
        out_shape=[jax.ShapeDtypeStruct((N, D), jnp.float32),
                   jax.ShapeDtypeStruct((N, 2 * D), jnp.float32),
                   jax.ShapeDtypeStruct((N, D), jnp.float32),
                   jax.ShapeDtypeStruct((N, D), jnp.float32)],
    )(x, posp, WinT, b_in, WlinT, WsrcT, WdstT, P1pT, pb1, P2T, pb2,
      A1T, ab1, A2T, ab2)


_sc_mesh = plsc.VectorSubcoreMesh(core_axis_name="c", subcore_axis_name="s")


@functools.partial(
    pl.kernel,
    mesh=_sc_mesh,
    out_type=[jax.ShapeDtypeStruct((EC, D), jnp.float32),
              jax.ShapeDtypeStruct((EC, 2 * D), jnp.float32)],
    scratch_types=[pltpu.VMEM((UNIT,), jnp.int32),
                   pltpu.VMEM((UNIT,), jnp.int32),
                   pltpu.VMEM((UNIT, D), jnp.float32),
                   pltpu.VMEM((UNIT, 2 * D), jnp.float32),
                   pltpu.SemaphoreType.DMA],
)
def _sc_gather(src1, dst1, T1, T2, outD, outS, idx_s, idx_d, bufD, bufS,
               sem):
    c = lax.axis_index("c")
    s = lax.axis_index("s")
    wid = s * NSC + c
    base = RC // NW
    nrows = base + jnp.where(wid < RC - base * NW, 1, 0)

    def body(i, carry):
        row = wid + i * NW
        off = pl.multiple_of(row * UNIT, UNIT)
        pltpu.sync_copy(src1.at[pl.ds(off, UNIT)], idx_s)
        pltpu.sync_copy(dst1.at[pl.ds(off, UNIT)], idx_d)
        cp1 = pltpu.async_copy(T1.at[idx_d], bufD, sem)
        cp2 = pltpu.async_copy(T2.at[idx_s], bufS, sem)
        cp1.wait()
        cp2.wait()
        pltpu.sync_copy(bufD, outD.at[pl.ds(off, UNIT), :])
        pltpu.sync_copy(bufS, outS.at[pl.ds(off, UNIT), :])
        return carry

    lax.fori_loop(0, nrows, body, 0)


def _emlp_body(eD_ref, eS_ref, pb1, P2T, pb2, A1T, ab1, A2T, ab2, out_ref):
    eD = eD_ref[...]
    eS = eS_ref[...]
    e1 = _relu(eD[:, :64] - eS[:, :64] + pb1[...])
    delta = _relu(jnp.dot(e1, P2T[...], preferred_element_type=jnp.float32)
                  + pb2[...])
    v = _relu(jnp.dot(delta, A1T[...], preferred_element_type=jnp.float32)
              + (eD[:, 64:] - eS[:, 64:128]) + ab1[...])
    alpha = _relu(jnp.dot(v, A2T[...], preferred_element_type=jnp.float32)
                  + ab2[...])
    ex = jnp.exp(alpha)
    msg = ex * (eS[:, 128:] + delta)
    out_ref[0] = jnp.concatenate([ex[:, :64], msg[:, :64]], axis=1)
    out_ref[1] = jnp.concatenate([ex[:, 64:], msg[:, 64:]], axis=1)


def _emlp_call(eD, eS, pb1, P2T, pb2, A1T, ab1, A2T, ab2):
    nb = EC // EPB
    full = lambda a: pl.BlockSpec(a.shape, lambda i: (0,) * a.ndim)
    return pl.pallas_call(
        _emlp_body,
        grid=(nb,),
        in_specs=[pl.BlockSpec((EPB, D), lambda i: (i, 0)),
                  pl.BlockSpec((EPB, 2 * D), lambda i: (i, 0))] + [
            full(a) for a in (pb1, P2T, pb2, A1T, ab1, A2T, ab2)],
        out_specs=pl.BlockSpec((2, EPB, D), lambda i: (0, i, 0)),
        out_shape=jax.ShapeDtypeStruct((2, EC, D), jnp.float32),
    )(eD, eS, pb1, P2T, pb2, A1T, ab1, A2T, ab2)


_RSLICE = 624


@functools.partial(
    pl.kernel,
    mesh=_sc_mesh,
    out_type=jax.ShapeDtypeStruct((2, N, D), jnp.float32),
    scratch_types=[pltpu.VMEM((UNIT,), jnp.int32),
                   pltpu.VMEM((UNIT, D), jnp.float32),
                   pltpu.VMEM_SHARED((N, D), jnp.float32)],
)
def _sc_scatter(dst1, em3, zeros, accO, idx_d, bufE, acc_sh):
    c = lax.axis_index("c")
    s = lax.axis_index("s")
    r0 = s * _RSLICE
    pltpu.sync_copy(zeros, acc_sh.at[pl.ds(r0, _RSLICE), :])

    @pl.when(s == NSUB - 1)
    def _():
        pltpu.sync_copy(zeros.at[pl.ds(0, 16), :],
                        acc_sh.at[pl.ds(NSUB * _RSLICE, 16), :])

    plsc.subcore_barrier()
    sbase = RC // NSUB
    nrows = sbase + jnp.where(s < RC - sbase * NSUB, 1, 0)

    def body(i, carry):
        row = s + i * NSUB
        off = pl.multiple_of(row * UNIT, UNIT)
        pltpu.sync_copy(dst1.at[pl.ds(off, UNIT)], idx_d)
        pltpu.sync_copy(em3.at[c, pl.ds(off, UNIT), :], bufE)
        pltpu.sync_copy(bufE, acc_sh.at[idx_d], add=True)
        return carry

    lax.fori_loop(0, nrows, body, 0)
    plsc.subcore_barrier()
    pltpu.sync_copy(acc_sh.at[pl.ds(r0, _RSLICE), :],
                    accO.at[c, pl.ds(r0, _RSLICE), :])

    @pl.when(s == NSUB - 1)
    def _():
        pltpu.sync_copy(acc_sh.at[pl.ds(NSUB * _RSLICE, 16), :],
                        accO.at[c, pl.ds(NSUB * _RSLICE, 16), :])


def _final_body(*refs):
    acc_refs = refs[:K]
    den0_ref, num0_ref, WoutT, b_out, o_ref = refs[K:]
    acc = acc_refs[0][...]
    for r in acc_refs[1:]:
        acc = acc + r[...]
    den = jnp.concatenate([acc[0, :, :64], acc[1, :, :64]], axis=1)
    num = jnp.concatenate([acc[0, :, 64:], acc[1, :, 64:]], axis=1)
    den = den + den0_ref[...] + 1e-16
    num = num + num0_ref[...]
    o_ref[...] = _relu(jnp.dot(num / den, WoutT[...],
                               preferred_element_type=jnp.float32)
                       + b_out[...])


def _final_call(accs, den0, num0, WoutT, b_out):
    nb = N // NPB
    full = lambda a: pl.BlockSpec(a.shape, lambda i: (0,) * a.ndim)
    row_spec = pl.BlockSpec((NPB, D), lambda i: (i, 0))
    acc_spec = pl.BlockSpec((2, NPB, D), lambda i: (0, i, 0))
    return pl.pallas_call(
        _final_body,
        grid=(nb,),
        in_specs=[acc_spec] * K + [row_spec, row_spec, full(WoutT),
                                   full(b_out)],
        out_specs=row_spec,
        out_shape=jax.ShapeDtypeStruct((N, D), jnp.float32),
    )(*accs, den0, num0, WoutT, b_out)


def kernel(x, pos, edge_index, Win, b_in, Wout, b_out, Wlin, Wsrc, Wdst,
           P1, pb1, P2, pb2, A1, ab1, A2, ab2):
    posp = jnp.pad(pos, ((0, 0), (0, 5)))
    P1pT = jnp.pad(P1, ((0, 0), (0, 5))).T
    row = lambda v: v.reshape(1, -1)
    T1, T2, den0, num0 = _prep_call(
        x, posp, Win.T, row(b_in), Wlin.T, Wsrc.T, Wdst.T, P1pT, row(pb1),
        P2.T, row(pb2), A1.T, row(ab1), A2.T, row(ab2))
    src1 = edge_index[0]
    dst1 = edge_index[1]
    zeros = jnp.zeros((_RSLICE, D), jnp.float32)
    accs = []
    for k in range(K):
        sl = slice(k * EC, (k + 1) * EC)
        eD, eS = _sc_gather(src1[sl], dst1[sl], T1, T2)
        em3 = _emlp_call(eD, eS, row(pb1), P2.T, row(pb2), A1.T, row(ab1),
                         A2.T, row(ab2))
        accs.append(_sc_scatter(dst1[sl], em3, zeros))
    return _final_call(accs, den0, num0, Wout.T, row(b_out))


# speedup vs baseline: 8.7782x; 1.0766x over previous
import functools

import jax
import jax.numpy as jnp
from jax import lax
from jax.experimental import pallas as pl
from jax.experimental.pallas import tpu as pltpu
from jax.experimental.pallas import tpu_sc as plsc

N = 10000
E = 320000
D = 128
UNIT = 128
R = E // UNIT
NSC = 2
NSUB = 16
NW = NSC * NSUB
NPB = 400
EPB = 1600
K = 2
EC = E // K
RC = EC // UNIT

_relu = jax.nn.relu


def _prep_body(x_ref, posp_ref, WinT, b_in, WlinT, WsrcT, WdstT, P1pT, pb1,
               P2T, pb2, A1T, ab1, A2T, ab2,
               T1_ref, T2_ref, den0_ref, num0_ref):
    x = x_ref[...]
    h = _relu(jnp.dot(x, WinT[...], preferred_element_type=jnp.float32)
              + b_in[...])
    xl = jnp.dot(h, WlinT[...], preferred_element_type=jnp.float32)
    dd = jnp.dot(jnp.dot(h, WdstT[...], preferred_element_type=jnp.float32),
                 A1T[...], preferred_element_type=jnp.float32)
    ss = jnp.dot(jnp.dot(h, WsrcT[...], preferred_element_type=jnp.float32),
                 A1T[...], preferred_element_type=jnp.float32)
    q = jnp.dot(posp_ref[...], P1pT[...], preferred_element_type=jnp.float32)
    dl64 = _relu(pb1[...])
    dl128 = _relu(jnp.dot(dl64, P2T[...],
                          preferred_element_type=jnp.float32) + pb2[...])
    v0 = _relu(jnp.dot(dl128, A1T[...], preferred_element_type=jnp.float32)
               + dd - ss + ab1[...])
    alpha0 = _relu(jnp.dot(v0, A2T[...], preferred_element_type=jnp.float32)
                   + ab2[...])
    ex0 = jnp.exp(alpha0)
    den0_ref[...] = ex0
    num0_ref[...] = ex0 * (xl + dl128)
    T1_ref[...] = jnp.concatenate([q, dd], axis=1)
    T2_ref[...] = jnp.concatenate([q, ss, xl], axis=1)


def _prep_call(x, posp, WinT, b_in, WlinT, WsrcT, WdstT, P1pT, pb1, P2T, pb2,
               A1T, ab1, A2T, ab2):
    nb = N // NPB
    full = lambda a: pl.BlockSpec(a.shape, lambda i: (0,) * a.ndim)
    row_spec = lambda w: pl.BlockSpec((NPB, w), lambda i: (i, 0))
    return pl.pallas_call(
        _prep_body,
        grid=(nb,),
        in_specs=[row_spec(D), row_spec(8)] + [
            full(a) for a in (WinT, b_in, WlinT, WsrcT, WdstT, P1pT, pb1,
                              P2T, pb2, A1T, ab1, A2T, ab2)],
        out_specs=[row_spec(D), row_spec(2 * D), row_spec(D), row_spec(D)],
        out_shape=[jax.ShapeDtypeStruct((N, D), jnp.float32),
                   jax.ShapeDtypeStruct((N, 2 * D), jnp.float32),
                   jax.ShapeDtypeStruct((N, D), jnp.float32),
                   jax.ShapeDtypeStruct((N, D), jnp.float32)],
    )(x, posp, WinT, b_in, WlinT, WsrcT, WdstT, P1pT, pb1, P2T, pb2,
      A1T, ab1, A2T, ab2)


_sc_mesh = plsc.VectorSubcoreMesh(core_axis_name="c", subcore_axis_name="s")


@functools.partial(
    pl.kernel,
    mesh=_sc_mesh,
    out_type=[jax.ShapeDtypeStruct((EC, D), jnp.float32),
              jax.ShapeDtypeStruct((EC, 2 * D), jnp.float32)],
    scratch_types=[pltpu.VMEM((UNIT,), jnp.int32),
                   pltpu.VMEM((UNIT,), jnp.int32),
                   pltpu.VMEM((UNIT, D), jnp.float32),
                   pltpu.VMEM((UNIT, 2 * D), jnp.float32),
                   pltpu.SemaphoreType.DMA],
)
def _sc_gather(src1, dst1, T1, T2, outD, outS, idx_s, idx_d, bufD, bufS,
               sem):
    c = lax.axis_index("c")
    s = lax.axis_index("s")
    wid = s * NSC + c
    base = RC // NW
    nrows = base + jnp.where(wid < RC - base * NW, 1, 0)

    def body(i, carry):
        row = wid + i * NW
        off = pl.multiple_of(row * UNIT, UNIT)
        pltpu.sync_copy(src1.at[pl.ds(off, UNIT)], idx_s)
        pltpu.sync_copy(dst1.at[pl.ds(off, UNIT)], idx_d)
        cp1 = pltpu.async_copy(T1.at[idx_d], bufD, sem)
        cp2 = pltpu.async_copy(T2.at[idx_s], bufS, sem)
        cp1.wait()
        cp2.wait()
        pltpu.sync_copy(bufD, outD.at[pl.ds(off, UNIT), :])
        pltpu.sync_copy(bufS, outS.at[pl.ds(off, UNIT), :])
        return carry

    lax.fori_loop(0, nrows, body, 0)


def _emlp_body(eD_ref, eS_ref, pb1, P2T, pb2, A1T, ab1, A2T, ab2, out_ref):
    eD = eD_ref[...]
    eS = eS_ref[...]
    e1 = _relu(eD[:, :64] - eS[:, :64] + pb1[...])
    delta = _relu(jnp.dot(e1, P2T[...], preferred_element_type=jnp.float32)
                  + pb2[...])
    v = _relu(jnp.dot(delta, A1T[...], preferred_element_type=jnp.float32)
              + (eD[:, 64:] - eS[:, 64:128]) + ab1[...])
    alpha = _relu(jnp.dot(v, A2T[...], preferred_element_type=jnp.float32)
                  + ab2[...])
    ex = jnp.exp(alpha)
    msg = ex * (eS[:, 128:] + delta)
    out_ref[0] = jnp.concatenate([ex[:, :64], msg[:, :64]], axis=1)
    out_ref[1] = jnp.concatenate([ex[:, 64:], msg[:, 64:]], axis=1)


def _emlp_call(eD, eS, pb1, P2T, pb2, A1T, ab1, A2T, ab2):
    nb = EC // EPB
    full = lambda a: pl.BlockSpec(a.shape, lambda i: (0,) * a.ndim)
    return pl.pallas_call(
        _emlp_body,
        grid=(nb,),
        in_specs=[pl.BlockSpec((EPB, D), lambda i: (i, 0)),
                  pl.BlockSpec((EPB, 2 * D), lambda i: (i, 0))] + [
            full(a) for a in (pb1, P2T, pb2, A1T, ab1, A2T, ab2)],
        out_specs=pl.BlockSpec((2, EPB, D), lambda i: (0, i, 0)),
        out_shape=jax.ShapeDtypeStruct((2, EC, D), jnp.float32),
    )(eD, eS, pb1, P2T, pb2, A1T, ab1, A2T, ab2)


_RSLICE = 624


@functools.partial(
    pl.kernel,
    mesh=_sc_mesh,
    out_type=jax.ShapeDtypeStruct((2, N, D), jnp.float32),
    scratch_types=[pltpu.VMEM((UNIT,), jnp.int32),
                   pltpu.VMEM((UNIT, D), jnp.float32),
                   pltpu.VMEM_SHARED((N, D), jnp.float32)],
)
def _sc_scatter(dst1, em3, zeros, accO, idx_d, bufE, acc_sh):
    c = lax.axis_index("c")
    s = lax.axis_index("s")
    r0 = s * _RSLICE
    pltpu.sync_copy(zeros, acc_sh.at[pl.ds(r0, _RSLICE), :])

    @pl.when(s == NSUB - 1)
    def _():
        pltpu.sync_copy(zeros.at[pl.ds(0, 16), :],
                        acc_sh.at[pl.ds(NSUB * _RSLICE, 16), :])

    plsc.subcore_barrier()
    sbase = RC // NSUB
    nrows = sbase + jnp.where(s < RC - sbase * NSUB, 1, 0)

    def body(i, carry):
        row = s + i * NSUB
        off = pl.multiple_of(row * UNIT, UNIT)
        pltpu.sync_copy(dst1.at[pl.ds(off, UNIT)], idx_d)
        pltpu.sync_copy(em3.at[c, pl.ds(off, UNIT), :], bufE)
        pltpu.sync_copy(bufE, acc_sh.at[idx_d], add=True)
        return carry

    lax.fori_loop(0, nrows, body, 0)
    plsc.subcore_barrier()
    pltpu.sync_copy(acc_sh.at[pl.ds(r0, _RSLICE), :],
                    accO.at[c, pl.ds(r0, _RSLICE), :])

    @pl.when(s == NSUB - 1)
    def _():
        pltpu.sync_copy(acc_sh.at[pl.ds(NSUB * _RSLICE, 16), :],
                        accO.at[c, pl.ds(NSUB * _RSLICE, 16), :])


def _final_body(*refs):
    acc_refs = refs[:K]
    den0_ref, num0_ref, WoutT, b_out, o_ref = refs[K:]
    acc = acc_refs[0][...]
    for r in acc_refs[1:]:
        acc = acc + r[...]
    den = jnp.concatenate([acc[0, :, :64], acc[1, :, :64]], axis=1)
    num = jnp.concatenate([acc[0, :, 64:], acc[1, :, 64:]], axis=1)
    den = den + den0_ref[...] + 1e-16
    num = num + num0_ref[...]
    o_ref[...] = _relu(jnp.dot(num / den, WoutT[...],
                               preferred_element_type=jnp.float32)
                       + b_out[...])


def _final_call(accs, den0, num0, WoutT, b_out):
    nb = N // NPB
    full = lambda a: pl.BlockSpec(a.shape, lambda i: (0,) * a.ndim)
    row_spec = pl.BlockSpec((NPB, D), lambda i: (i, 0))
    acc_spec = pl.BlockSpec((2, NPB, D), lambda i: (0, i, 0))
    return pl.pallas_call(
        _final_body,
        grid=(nb,),
        in_specs=[acc_spec] * K + [row_spec, row_spec, full(WoutT),
                                   full(b_out)],
        out_specs=row_spec,
        out_shape=jax.ShapeDtypeStruct((N, D), jnp.float32),
    )(*accs, den0, num0, WoutT, b_out)


def kernel(x, pos, edge_index, Win, b_in, Wout, b_out, Wlin, Wsrc, Wdst,
           P1, pb1, P2, pb2, A1, ab1, A2, ab2):
    posp = jnp.pad(pos, ((0, 0), (0, 5)))
    P1pT = jnp.pad(P1, ((0, 0), (0, 5))).T
    row = lambda v: v.reshape(1, -1)
    T1, T2, den0, num0 = _prep_call(
        x, posp, Win.T, row(b_in), Wlin.T, Wsrc.T, Wdst.T, P1pT, row(pb1),
        P2.T, row(pb2), A1.T, row(ab1), A2.T, row(ab2))
    src1 = edge_index[0]
    dst1 = edge_index[1]
    zeros = jnp.zeros((_RSLICE, D), jnp.float32)
    accs = []
    for k in range(K):
        sl = slice(k * EC, (k + 1) * EC)
        eD, eS = _sc_gather(src1[sl], dst1[sl], T1, T2)
        em3 = _emlp_call(eD, eS, row(pb1), P2.T, row(pb2), A1.T, row(ab1),
                         A2.T, row(ab2))
        accs.append(_sc_scatter(dst1[sl], em3, zeros))
    return _final_call(accs, den0, num0, Wout.T, row(b_out))
